# baseline (speedup 1.0000x reference)
import functools

import jax
import jax.numpy as jnp
from jax import lax
from jax.experimental import pallas as pl
from jax.experimental.pallas import tpu as pltpu
from jax.experimental.pallas import tpu_sc as plsc

N = 10000
N_PAD = 10240
E = 320000
HID = 128
OUT_CH = 40
BN_EPS = 1e-5

NC = 2
NS = 16
NW = NC * NS
CHUNK = 64
NCHUNK = 160
E_PER_W = NCHUNK * CHUNK
E_PAD = NW * E_PER_W
BLK = 5
NBLK = NCHUNK // BLK
ROWS_PER_TILE = N_PAD // NS


def _sc_mesh():
    return plsc.VectorSubcoreMesh(core_axis_name="c", subcore_axis_name="s")


@functools.partial(
    pl.kernel,
    out_type=jax.ShapeDtypeStruct((NC, N_PAD, HID), jnp.float32),
    mesh=_sc_mesh(),
    scratch_types=[
        pltpu.VMEM((2, BLK, CHUNK), jnp.int32),
        pltpu.VMEM((2, BLK, CHUNK), jnp.int32),
        pltpu.VMEM((4, CHUNK, HID), jnp.float32),
        pltpu.VMEM_SHARED((N_PAD, HID), jnp.float32),
        pltpu.SemaphoreType.DMA((2,)),
        pltpu.SemaphoreType.DMA((4,)),
        pltpu.SemaphoreType.DMA((4,)),
    ],
)
def _sc_aggregate(h_hbm, src_hbm, dst_hbm, zero_hbm, out_hbm,
                  src_v, dst_v, rows_v, agg_sh, isem, gsem, ssem):
    c = lax.axis_index("c")
    s = lax.axis_index("s")
    wid = c * NS + s

    pltpu.async_copy(src_hbm.at[wid, 0], src_v.at[0], isem.at[0])
    pltpu.async_copy(dst_hbm.at[wid, 0], dst_v.at[0], isem.at[0])
    r0 = s * ROWS_PER_TILE
    pltpu.sync_copy(zero_hbm.at[pl.ds(r0, ROWS_PER_TILE)],
                    agg_sh.at[pl.ds(r0, ROWS_PER_TILE)])
    plsc.subcore_barrier()
    pltpu.make_async_copy(src_hbm.at[wid, 0], src_v.at[0], isem.at[0]).wait()
    pltpu.make_async_copy(dst_hbm.at[wid, 0], dst_v.at[0], isem.at[0]).wait()
    pltpu.async_copy(src_hbm.at[wid, 1], src_v.at[1], isem.at[1])
    pltpu.async_copy(dst_hbm.at[wid, 1], dst_v.at[1], isem.at[1])
    pltpu.async_copy(h_hbm.at[src_v.at[0, 0]], rows_v.at[0], gsem.at[0])
    pltpu.async_copy(h_hbm.at[src_v.at[0, 1]], rows_v.at[1], gsem.at[1])

    def body(i, carry):
        b = i % 4
        blk = i // BLK
        sl = blk % 2
        j = i - blk * BLK
        pltpu.make_async_copy(h_hbm.at[src_v.at[sl, j]], rows_v.at[b],
                              gsem.at[b]).wait()
        pltpu.async_copy(rows_v.at[b], agg_sh.at[dst_v.at[sl, j]],
                         ssem.at[b], add=True)

        @pl.when(i > 1)
        def _():
            pltpu.make_async_copy(rows_v.at[(i + 2) % 4],
                                  agg_sh.at[dst_v.at[sl, j]],
                                  ssem.at[(i + 2) % 4]).wait()

        @pl.when(jnp.logical_and(j == 0,
                                 jnp.logical_and(i > 0, blk + 1 < NBLK)))
        def _():
            nsl = (blk + 1) % 2
            pltpu.async_copy(src_hbm.at[wid, blk + 1], src_v.at[nsl],
                             isem.at[nsl])
            pltpu.async_copy(dst_hbm.at[wid, blk + 1], dst_v.at[nsl],
                             isem.at[nsl])

        t = i + 2

        @pl.when(t < NCHUNK)
        def _():
            tb = t // BLK
            tsl = tb % 2
            tj = t - tb * BLK

            @pl.when(tj == 0)
            def _():
                pltpu.make_async_copy(src_hbm.at[wid, tb], src_v.at[tsl],
                                      isem.at[tsl]).wait()
                pltpu.make_async_copy(dst_hbm.at[wid, tb], dst_v.at[tsl],
                                      isem.at[tsl]).wait()
            pltpu.async_copy(h_hbm.at[src_v.at[tsl, tj]], rows_v.at[t % 4],
                             gsem.at[t % 4])
        return carry

    lax.fori_loop(0, NCHUNK, body, 0, unroll=False)
    pltpu.make_async_copy(rows_v.at[(NCHUNK - 2) % 4],
                          agg_sh.at[dst_v.at[0, 0]],
                          ssem.at[(NCHUNK - 2) % 4]).wait()
    pltpu.make_async_copy(rows_v.at[(NCHUNK - 1) % 4],
                          agg_sh.at[dst_v.at[0, 0]],
                          ssem.at[(NCHUNK - 1) % 4]).wait()
    plsc.subcore_barrier()

    pltpu.sync_copy(agg_sh.at[pl.ds(r0, ROWS_PER_TILE)],
                    out_hbm.at[c, pl.ds(r0, ROWS_PER_TILE)])


def _mlp_body(h_ref, p_ref, eps_ref, w1_ref, b1_ref, w2_ref, b2_ref, o_ref):
    z = h_ref[...] * (1.0 + eps_ref[0]) + p_ref[0] + p_ref[1]
    z1 = jnp.dot(z, w1_ref[...], preferred_element_type=jnp.float32) + b1_ref[...]
    z1 = jnp.maximum(z1, 0.0)
    z2 = jnp.dot(z1, w2_ref[...], preferred_element_type=jnp.float32) + b2_ref[...]
    o_ref[...] = jnp.maximum(z2, 0.0)


def _mlp_head_body(h_ref, p_ref, eps_ref, w1_ref, b1_ref, w2_ref, b2_ref,
                   wa_ref, ba_ref, wb_ref, bb_ref, o_ref):
    z = h_ref[...] * (1.0 + eps_ref[0]) + p_ref[0] + p_ref[1]
    z1 = jnp.dot(z, w1_ref[...], preferred_element_type=jnp.float32) + b1_ref[...]
    z1 = jnp.maximum(z1, 0.0)
    z2 = jnp.dot(z1, w2_ref[...], preferred_element_type=jnp.float32) + b2_ref[...]
    h3 = jnp.maximum(z2, 0.0)
    y1 = jnp.dot(h3, wa_ref[...], preferred_element_type=jnp.float32)
    y1 = jnp.maximum(y1 + ba_ref[...], 0.0)
    y2 = jnp.dot(y1, wb_ref[...], preferred_element_type=jnp.float32) + bb_ref[...]
    m = jnp.max(y2, axis=-1, keepdims=True)
    e = jnp.exp(y2 - m)
    o_ref[...] = y2 - m - jnp.log(jnp.sum(e, axis=-1, keepdims=True))


_BM = 2048


def _run_mlp(h, part, eps, w1, b1, w2, b2):
    grid = N_PAD // _BM
    return pl.pallas_call(
        _mlp_body,
        grid=(grid,),
        in_specs=[
            pl.BlockSpec((_BM, HID), lambda i: (i, 0)),
            pl.BlockSpec((NC, _BM, HID), lambda i: (0, i, 0)),
            pl.BlockSpec(memory_space=pltpu.SMEM),
            pl.BlockSpec((HID, 2 * HID), lambda i: (0, 0)),
            pl.BlockSpec((1, 2 * HID), lambda i: (0, 0)),
            pl.BlockSpec((2 * HID, HID), lambda i: (0, 0)),
            pl.BlockSpec((1, HID), lambda i: (0, 0)),
        ],
        out_specs=pl.BlockSpec((_BM, HID), lambda i: (i, 0)),
        out_shape=jax.ShapeDtypeStruct((N_PAD, HID), jnp.float32),
    )(h, part, eps, w1, b1, w2, b2)


def _run_mlp_head(h, part, eps, w1, b1, w2, b2, wa, ba, wb, bb):
    grid = N_PAD // _BM
    return pl.pallas_call(
        _mlp_head_body,
        grid=(grid,),
        in_specs=[
            pl.BlockSpec((_BM, HID), lambda i: (i, 0)),
            pl.BlockSpec((NC, _BM, HID), lambda i: (0, i, 0)),
            pl.BlockSpec(memory_space=pltpu.SMEM),
            pl.BlockSpec((HID, 2 * HID), lambda i: (0, 0)),
            pl.BlockSpec((1, 2 * HID), lambda i: (0, 0)),
            pl.BlockSpec((2 * HID, HID), lambda i: (0, 0)),
            pl.BlockSpec((1, HID), lambda i: (0, 0)),
            pl.BlockSpec((HID, HID), lambda i: (0, 0)),
            pl.BlockSpec((1, HID), lambda i: (0, 0)),
            pl.BlockSpec((HID, OUT_CH), lambda i: (0, 0)),
            pl.BlockSpec((1, OUT_CH), lambda i: (0, 0)),
        ],
        out_specs=pl.BlockSpec((_BM, OUT_CH), lambda i: (i, 0)),
        out_shape=jax.ShapeDtypeStruct((N_PAD, OUT_CH), jnp.float32),
    )(h, part, eps, w1, b1, w2, b2, wa, ba, wb, bb)


def kernel(x, edge_index, params):
    inv = 1.0 / jnp.sqrt(jnp.float32(1.0 + BN_EPS))

    src3 = jnp.concatenate(
        [edge_index[0], jnp.zeros((E_PAD - E,), jnp.int32)]
    ).reshape(NW, NBLK, BLK, CHUNK)
    dst3 = jnp.concatenate(
        [edge_index[1], jnp.full((E_PAD - E,), N, jnp.int32)]
    ).reshape(NW, NBLK, BLK, CHUNK)
    zeros = jnp.zeros((N_PAD, HID), jnp.float32)

    sa = params['bn1_g'] * inv
    wa = params['lin1_W'] * sa[None, :]
    ba = (params['lin1_b'] * sa + params['bn1_b'])[None, :]
    wb = params['lin2_W']
    bb = params['lin2_b'][None, :]

    h = jnp.pad(x, ((0, N_PAD - N), (0, 0)))
    for li, layer in enumerate(params['convs']):
        s1 = layer['mlp_bn_g'] * inv
        w1 = layer['W1'] * s1[None, :]
        b1 = (layer['b1'] * s1 + layer['mlp_bn_b'])[None, :]
        s2 = layer['out_bn_g'] * inv
        w2 = layer['W2'] * s2[None, :]
        b2 = (layer['b2'] * s2 + layer['out_bn_b'])[None, :]
        eps = layer['eps'].reshape(1)

        part = _sc_aggregate(h, src3, dst3, zeros)
        if li < len(params['convs']) - 1:
            h = _run_mlp(h, part, eps, w1, b1, w2, b2)
        else:
            out = _run_mlp_head(h, part, eps, w1, b1, w2, b2, wa, ba, wb, bb)
    return out[:N]


# speedup vs baseline: 1.0005x; 1.0005x over previous
import functools

import jax
import jax.numpy as jnp
from jax import lax
from jax.experimental import pallas as pl
from jax.experimental.pallas import tpu as pltpu
from jax.experimental.pallas import tpu_sc as plsc

N = 10000
N_PAD = 10240
E = 320000
HID = 128
OUT_CH = 40
BN_EPS = 1e-5

NC = 2
NS = 16
NW = NC * NS
CHUNK = 64
NCHUNK = 160
E_PER_W = NCHUNK * CHUNK
E_PAD = NW * E_PER_W
BLK = 5
NBLK = NCHUNK // BLK
ROWS_PER_TILE = N_PAD // NS


def _sc_mesh():
    return plsc.VectorSubcoreMesh(core_axis_name="c", subcore_axis_name="s")


@functools.partial(
    pl.kernel,
    out_type=jax.ShapeDtypeStruct((NC, N_PAD, HID), jnp.float32),
    mesh=_sc_mesh(),
    scratch_types=[
        pltpu.VMEM((2, BLK, CHUNK), jnp.int32),
        pltpu.VMEM((2, BLK, CHUNK), jnp.int32),
        pltpu.VMEM((4, CHUNK, HID), jnp.float32),
        pltpu.VMEM_SHARED((N_PAD, HID), jnp.float32),
        pltpu.SemaphoreType.DMA((2,)),
        pltpu.SemaphoreType.DMA((4,)),
        pltpu.SemaphoreType.DMA((4,)),
    ],
)
def _sc_aggregate(h_hbm, src_hbm, dst_hbm, zero_hbm, out_hbm,
                  src_v, dst_v, rows_v, agg_sh, isem, gsem, ssem):
    c = lax.axis_index("c")
    s = lax.axis_index("s")
    wid = c * NS + s

    pltpu.async_copy(src_hbm.at[wid, 0], src_v.at[0], isem.at[0])
    pltpu.async_copy(dst_hbm.at[wid, 0], dst_v.at[0], isem.at[0])
    r0 = s * ROWS_PER_TILE
    pltpu.sync_copy(zero_hbm.at[pl.ds(r0, ROWS_PER_TILE)],
                    agg_sh.at[pl.ds(r0, ROWS_PER_TILE)])
    plsc.subcore_barrier()
    pltpu.make_async_copy(src_hbm.at[wid, 0], src_v.at[0], isem.at[0]).wait()
    pltpu.make_async_copy(dst_hbm.at[wid, 0], dst_v.at[0], isem.at[0]).wait()
    pltpu.async_copy(src_hbm.at[wid, 1], src_v.at[1], isem.at[1])
    pltpu.async_copy(dst_hbm.at[wid, 1], dst_v.at[1], isem.at[1])
    pltpu.async_copy(h_hbm.at[src_v.at[0, 0]], rows_v.at[0], gsem.at[0])
    pltpu.async_copy(h_hbm.at[src_v.at[0, 1]], rows_v.at[1], gsem.at[1])

    def body(i, carry):
        b = i % 4
        blk = i // BLK
        sl = blk % 2
        j = i - blk * BLK
        pltpu.make_async_copy(h_hbm.at[src_v.at[sl, j]], rows_v.at[b],
                              gsem.at[b]).wait()
        pltpu.async_copy(rows_v.at[b], agg_sh.at[dst_v.at[sl, j]],
                         ssem.at[b], add=True)

        @pl.when(i > 1)
        def _():
            pltpu.make_async_copy(rows_v.at[(i + 2) % 4],
                                  agg_sh.at[dst_v.at[sl, j]],
                                  ssem.at[(i + 2) % 4]).wait()

        @pl.when(jnp.logical_and(j == 0,
                                 jnp.logical_and(i > 0, blk + 1 < NBLK)))
        def _():
            nsl = (blk + 1) % 2
            pltpu.async_copy(src_hbm.at[wid, blk + 1], src_v.at[nsl],
                             isem.at[nsl])
            pltpu.async_copy(dst_hbm.at[wid, blk + 1], dst_v.at[nsl],
                             isem.at[nsl])

        t = i + 2

        @pl.when(t < NCHUNK)
        def _():
            tb = t // BLK
            tsl = tb % 2
            tj = t - tb * BLK

            @pl.when(tj == 0)
            def _():
                pltpu.make_async_copy(src_hbm.at[wid, tb], src_v.at[tsl],
                                      isem.at[tsl]).wait()
                pltpu.make_async_copy(dst_hbm.at[wid, tb], dst_v.at[tsl],
                                      isem.at[tsl]).wait()
            pltpu.async_copy(h_hbm.at[src_v.at[tsl, tj]], rows_v.at[t % 4],
                             gsem.at[t % 4])
        return carry

    lax.fori_loop(0, NCHUNK, body, 0, unroll=False)
    pltpu.make_async_copy(rows_v.at[(NCHUNK - 2) % 4],
                          agg_sh.at[dst_v.at[0, 0]],
                          ssem.at[(NCHUNK - 2) % 4]).wait()
    pltpu.make_async_copy(rows_v.at[(NCHUNK - 1) % 4],
                          agg_sh.at[dst_v.at[0, 0]],
                          ssem.at[(NCHUNK - 1) % 4]).wait()
    plsc.subcore_barrier()

    pltpu.sync_copy(agg_sh.at[pl.ds(r0, ROWS_PER_TILE)],
                    out_hbm.at[c, pl.ds(r0, ROWS_PER_TILE)])


def _mlp_body(h_ref, p_ref, eps_ref, w1_ref, b1_ref, w2_ref, b2_ref, o_ref):
    z = h_ref[...] * (1.0 + eps_ref[0]) + p_ref[0] + p_ref[1]
    z1 = jnp.dot(z, w1_ref[...], preferred_element_type=jnp.float32) + b1_ref[...]
    z1 = jnp.maximum(z1, 0.0)
    z2 = jnp.dot(z1, w2_ref[...], preferred_element_type=jnp.float32) + b2_ref[...]
    o_ref[...] = jnp.maximum(z2, 0.0)


def _mlp_head_body(h_ref, p_ref, eps_ref, w1_ref, b1_ref, w2_ref, b2_ref,
                   wa_ref, ba_ref, wb_ref, bb_ref, o_ref):
    z = h_ref[...] * (1.0 + eps_ref[0]) + p_ref[0] + p_ref[1]
    z1 = jnp.dot(z, w1_ref[...], preferred_element_type=jnp.float32) + b1_ref[...]
    z1 = jnp.maximum(z1, 0.0)
    z2 = jnp.dot(z1, w2_ref[...], preferred_element_type=jnp.float32) + b2_ref[...]
    h3 = jnp.maximum(z2, 0.0)
    y1 = jnp.dot(h3, wa_ref[...], preferred_element_type=jnp.float32)
    y1 = jnp.maximum(y1 + ba_ref[...], 0.0)
    y2 = jnp.dot(y1, wb_ref[...], preferred_element_type=jnp.float32) + bb_ref[...]
    m = jnp.max(y2, axis=-1, keepdims=True)
    e = jnp.exp(y2 - m)
    o_ref[...] = y2 - m - jnp.log(jnp.sum(e, axis=-1, keepdims=True))


_BM = 2048


def _run_mlp(h, part, eps, w1, b1, w2, b2):
    grid = N_PAD // _BM
    return pl.pallas_call(
        _mlp_body,
        grid=(grid,),
        in_specs=[
            pl.BlockSpec((_BM, HID), lambda i: (i, 0)),
            pl.BlockSpec((NC, _BM, HID), lambda i: (0, i, 0)),
            pl.BlockSpec(memory_space=pltpu.SMEM),
            pl.BlockSpec((HID, 2 * HID), lambda i: (0, 0)),
            pl.BlockSpec((1, 2 * HID), lambda i: (0, 0)),
            pl.BlockSpec((2 * HID, HID), lambda i: (0, 0)),
            pl.BlockSpec((1, HID), lambda i: (0, 0)),
        ],
        out_specs=pl.BlockSpec((_BM, HID), lambda i: (i, 0)),
        out_shape=jax.ShapeDtypeStruct((N_PAD, HID), jnp.float32),
    )(h, part, eps, w1, b1, w2, b2)


def _run_mlp_head(h, part, eps, w1, b1, w2, b2, wa, ba, wb, bb):
    grid = N_PAD // _BM
    return pl.pallas_call(
        _mlp_head_body,
        grid=(grid,),
        in_specs=[
            pl.BlockSpec((_BM, HID), lambda i: (i, 0)),
            pl.BlockSpec((NC, _BM, HID), lambda i: (0, i, 0)),
            pl.BlockSpec(memory_space=pltpu.SMEM),
            pl.BlockSpec((HID, 2 * HID), lambda i: (0, 0)),
            pl.BlockSpec((1, 2 * HID), lambda i: (0, 0)),
            pl.BlockSpec((2 * HID, HID), lambda i: (0, 0)),
            pl.BlockSpec((1, HID), lambda i: (0, 0)),
            pl.BlockSpec((HID, HID), lambda i: (0, 0)),
            pl.BlockSpec((1, HID), lambda i: (0, 0)),
            pl.BlockSpec((HID, OUT_CH), lambda i: (0, 0)),
            pl.BlockSpec((1, OUT_CH), lambda i: (0, 0)),
        ],
        out_specs=pl.BlockSpec((_BM, OUT_CH), lambda i: (i, 0)),
        out_shape=jax.ShapeDtypeStruct((N_PAD, OUT_CH), jnp.float32),
    )(h, part, eps, w1, b1, w2, b2, wa, ba, wb, bb)


def kernel(x, edge_index, params):
    inv = 1.0 / jnp.sqrt(jnp.float32(1.0 + BN_EPS))

    src3 = jnp.concatenate(
        [edge_index[0], jnp.zeros((E_PAD - E,), jnp.int32)]
    ).reshape(NW, NBLK, BLK, CHUNK)
    pad_dst = N + jnp.arange(E_PAD - E, dtype=jnp.int32) % (N_PAD - N)
    dst3 = jnp.concatenate(
        [edge_index[1], pad_dst]
    ).reshape(NW, NBLK, BLK, CHUNK)
    zeros = jnp.zeros((N_PAD, HID), jnp.float32)

    sa = params['bn1_g'] * inv
    wa = params['lin1_W'] * sa[None, :]
    ba = (params['lin1_b'] * sa + params['bn1_b'])[None, :]
    wb = params['lin2_W']
    bb = params['lin2_b'][None, :]

    h = jnp.pad(x, ((0, N_PAD - N), (0, 0)))
    for li, layer in enumerate(params['convs']):
        s1 = layer['mlp_bn_g'] * inv
        w1 = layer['W1'] * s1[None, :]
        b1 = (layer['b1'] * s1 + layer['mlp_bn_b'])[None, :]
        s2 = layer['out_bn_g'] * inv
        w2 = layer['W2'] * s2[None, :]
        b2 = (layer['b2'] * s2 + layer['out_bn_b'])[None, :]
        eps = layer['eps'].reshape(1)

        part = _sc_aggregate(h, src3, dst3, zeros)
        if li < len(params['convs']) - 1:
            h = _run_mlp(h, part, eps, w1, b1, w2, b2)
        else:
            out = _run_mlp_head(h, part, eps, w1, b1, w2, b2, wa, ba, wb, bb)
    return out[:N]


# speedup vs baseline: 3.9044x; 3.9024x over previous
import functools

import jax
import jax.numpy as jnp
from jax import lax
from jax.experimental import pallas as pl
from jax.experimental.pallas import tpu as pltpu
from jax.experimental.pallas import tpu_sc as plsc

N = 10000
N_PAD = 10240
E = 320000
HID = 128
OUT_CH = 40
BN_EPS = 1e-5

NC = 2
NS = 16
NW = NC * NS
E_PER_W = E // NW
CHUNK = 80
NCHUNK = E_PER_W // CHUNK
BLK = 5
NBLK = NCHUNK // BLK
ROWS_PER_TILE = N_PAD // NS


def _sc_mesh():
    return plsc.VectorSubcoreMesh(core_axis_name="c", subcore_axis_name="s")


@functools.partial(
    pl.kernel,
    out_type=jax.ShapeDtypeStruct((NC, N_PAD, HID), jnp.float32),
    mesh=_sc_mesh(),
    scratch_types=[
        pltpu.VMEM((2, BLK, CHUNK), jnp.int32),
        pltpu.VMEM((2, BLK, CHUNK), jnp.int32),
        pltpu.VMEM((3, CHUNK, HID), jnp.float32),
        pltpu.VMEM_SHARED((N_PAD, HID), jnp.float32),
        pltpu.SemaphoreType.DMA((2,)),
        pltpu.SemaphoreType.DMA((3,)),
        pltpu.SemaphoreType.DMA((3,)),
    ],
)
def _sc_aggregate(h_hbm, src_hbm, dst_hbm, zero_hbm, out_hbm,
                  src_v, dst_v, rows_v, agg_sh, isem, gsem, ssem):
    c = lax.axis_index("c")
    s = lax.axis_index("s")
    wid = c * NS + s

    pltpu.async_copy(src_hbm.at[wid, 0], src_v.at[0], isem.at[0])
    pltpu.async_copy(dst_hbm.at[wid, 0], dst_v.at[0], isem.at[0])
    r0 = s * ROWS_PER_TILE
    pltpu.sync_copy(zero_hbm.at[pl.ds(r0, ROWS_PER_TILE)],
                    agg_sh.at[pl.ds(r0, ROWS_PER_TILE)])
    plsc.subcore_barrier()
    pltpu.make_async_copy(src_hbm.at[wid, 0], src_v.at[0], isem.at[0]).wait()
    pltpu.make_async_copy(dst_hbm.at[wid, 0], dst_v.at[0], isem.at[0]).wait()
    pltpu.async_copy(src_hbm.at[wid, 1], src_v.at[1], isem.at[1])
    pltpu.async_copy(dst_hbm.at[wid, 1], dst_v.at[1], isem.at[1])
    pltpu.async_copy(h_hbm.at[src_v.at[0, 0]], rows_v.at[0], gsem.at[0])
    pltpu.async_copy(h_hbm.at[src_v.at[0, 1]], rows_v.at[1], gsem.at[1])

    def body(i, carry):
        b = i % 3
        blk = i // BLK
        sl = blk % 2
        j = i - blk * BLK
        pltpu.make_async_copy(h_hbm.at[src_v.at[sl, j]], rows_v.at[b],
                              gsem.at[b]).wait()
        pltpu.async_copy(rows_v.at[b], agg_sh.at[dst_v.at[sl, j]],
                         ssem.at[b], add=True)

        @pl.when(i > 0)
        def _():
            pltpu.make_async_copy(rows_v.at[(i + 2) % 3],
                                  agg_sh.at[dst_v.at[sl, j]],
                                  ssem.at[(i + 2) % 3]).wait()

        @pl.when(jnp.logical_and(j == 0,
                                 jnp.logical_and(i > 0, blk + 1 < NBLK)))
        def _():
            nsl = (blk + 1) % 2
            pltpu.async_copy(src_hbm.at[wid, blk + 1], src_v.at[nsl],
                             isem.at[nsl])
            pltpu.async_copy(dst_hbm.at[wid, blk + 1], dst_v.at[nsl],
                             isem.at[nsl])

        t = i + 2

        @pl.when(t < NCHUNK)
        def _():
            tb = t // BLK
            tsl = tb % 2
            tj = t - tb * BLK

            @pl.when(tj == 0)
            def _():
                pltpu.make_async_copy(src_hbm.at[wid, tb], src_v.at[tsl],
                                      isem.at[tsl]).wait()
                pltpu.make_async_copy(dst_hbm.at[wid, tb], dst_v.at[tsl],
                                      isem.at[tsl]).wait()
            pltpu.async_copy(h_hbm.at[src_v.at[tsl, tj]], rows_v.at[t % 3],
                             gsem.at[t % 3])
        return carry

    lax.fori_loop(0, NCHUNK, body, 0, unroll=False)
    pltpu.make_async_copy(rows_v.at[(NCHUNK - 1) % 3],
                          agg_sh.at[dst_v.at[0, 0]],
                          ssem.at[(NCHUNK - 1) % 3]).wait()
    plsc.subcore_barrier()

    pltpu.sync_copy(agg_sh.at[pl.ds(r0, ROWS_PER_TILE)],
                    out_hbm.at[c, pl.ds(r0, ROWS_PER_TILE)])


def _mlp_body(h_ref, p_ref, eps_ref, w1_ref, b1_ref, w2_ref, b2_ref, o_ref):
    z = h_ref[...] * (1.0 + eps_ref[0]) + p_ref[0] + p_ref[1]
    z1 = jnp.dot(z, w1_ref[...], preferred_element_type=jnp.float32) + b1_ref[...]
    z1 = jnp.maximum(z1, 0.0)
    z2 = jnp.dot(z1, w2_ref[...], preferred_element_type=jnp.float32) + b2_ref[...]
    o_ref[...] = jnp.maximum(z2, 0.0)


def _mlp_head_body(h_ref, p_ref, eps_ref, w1_ref, b1_ref, w2_ref, b2_ref,
                   wa_ref, ba_ref, wb_ref, bb_ref, o_ref):
    z = h_ref[...] * (1.0 + eps_ref[0]) + p_ref[0] + p_ref[1]
    z1 = jnp.dot(z, w1_ref[...], preferred_element_type=jnp.float32) + b1_ref[...]
    z1 = jnp.maximum(z1, 0.0)
    z2 = jnp.dot(z1, w2_ref[...], preferred_element_type=jnp.float32) + b2_ref[...]
    h3 = jnp.maximum(z2, 0.0)
    y1 = jnp.dot(h3, wa_ref[...], preferred_element_type=jnp.float32)
    y1 = jnp.maximum(y1 + ba_ref[...], 0.0)
    y2 = jnp.dot(y1, wb_ref[...], preferred_element_type=jnp.float32) + bb_ref[...]
    m = jnp.max(y2, axis=-1, keepdims=True)
    e = jnp.exp(y2 - m)
    o_ref[...] = y2 - m - jnp.log(jnp.sum(e, axis=-1, keepdims=True))


_BM = 2048


def _run_mlp(h, part, eps, w1, b1, w2, b2):
    grid = N_PAD // _BM
    return pl.pallas_call(
        _mlp_body,
        grid=(grid,),
        in_specs=[
            pl.BlockSpec((_BM, HID), lambda i: (i, 0)),
            pl.BlockSpec((NC, _BM, HID), lambda i: (0, i, 0)),
            pl.BlockSpec(memory_space=pltpu.SMEM),
            pl.BlockSpec((HID, 2 * HID), lambda i: (0, 0)),
            pl.BlockSpec((1, 2 * HID), lambda i: (0, 0)),
            pl.BlockSpec((2 * HID, HID), lambda i: (0, 0)),
            pl.BlockSpec((1, HID), lambda i: (0, 0)),
        ],
        out_specs=pl.BlockSpec((_BM, HID), lambda i: (i, 0)),
        out_shape=jax.ShapeDtypeStruct((N_PAD, HID), jnp.float32),
    )(h, part, eps, w1, b1, w2, b2)


def _run_mlp_head(h, part, eps, w1, b1, w2, b2, wa, ba, wb, bb):
    grid = N_PAD // _BM
    return pl.pallas_call(
        _mlp_head_body,
        grid=(grid,),
        in_specs=[
            pl.BlockSpec((_BM, HID), lambda i: (i, 0)),
            pl.BlockSpec((NC, _BM, HID), lambda i: (0, i, 0)),
            pl.BlockSpec(memory_space=pltpu.SMEM),
            pl.BlockSpec((HID, 2 * HID), lambda i: (0, 0)),
            pl.BlockSpec((1, 2 * HID), lambda i: (0, 0)),
            pl.BlockSpec((2 * HID, HID), lambda i: (0, 0)),
            pl.BlockSpec((1, HID), lambda i: (0, 0)),
            pl.BlockSpec((HID, HID), lambda i: (0, 0)),
            pl.BlockSpec((1, HID), lambda i: (0, 0)),
            pl.BlockSpec((HID, OUT_CH), lambda i: (0, 0)),
            pl.BlockSpec((1, OUT_CH), lambda i: (0, 0)),
        ],
        out_specs=pl.BlockSpec((_BM, OUT_CH), lambda i: (i, 0)),
        out_shape=jax.ShapeDtypeStruct((N_PAD, OUT_CH), jnp.float32),
    )(h, part, eps, w1, b1, w2, b2, wa, ba, wb, bb)


def kernel(x, edge_index, params):
    inv = 1.0 / jnp.sqrt(jnp.float32(1.0 + BN_EPS))

    src3 = edge_index[0].reshape(NW, NBLK, BLK, CHUNK)
    dst3 = edge_index[1].reshape(NW, NBLK, BLK, CHUNK)
    zeros = jnp.zeros((N_PAD, HID), jnp.float32)

    sa = params['bn1_g'] * inv
    wa = params['lin1_W'] * sa[None, :]
    ba = (params['lin1_b'] * sa + params['bn1_b'])[None, :]
    wb = params['lin2_W']
    bb = params['lin2_b'][None, :]

    h = jnp.pad(x, ((0, N_PAD - N), (0, 0)))
    for li, layer in enumerate(params['convs']):
        s1 = layer['mlp_bn_g'] * inv
        w1 = layer['W1'] * s1[None, :]
        b1 = (layer['b1'] * s1 + layer['mlp_bn_b'])[None, :]
        s2 = layer['out_bn_g'] * inv
        w2 = layer['W2'] * s2[None, :]
        b2 = (layer['b2'] * s2 + layer['out_bn_b'])[None, :]
        eps = layer['eps'].reshape(1)

        part = _sc_aggregate(h, src3, dst3, zeros)
        if li < len(params['convs']) - 1:
            h = _run_mlp(h, part, eps, w1, b1, w2, b2)
        else:
            out = _run_mlp_head(h, part, eps, w1, b1, w2, b2, wa, ba, wb, bb)
    return out[:N]
